# baseline (speedup 1.0000x reference)
import functools

import jax
import jax.numpy as jnp
from jax.experimental import pallas as pl
from jax.experimental.pallas import tpu as pltpu

_DIM = 64
_MAX_LENGTH = 160
_TBL = 2 * _MAX_LENGTH


def _body(tbl_ref, out_ref, ctv, *, seq):
    pid = pl.program_id(0)

    @pl.when(pid == 0)
    def _build():
        crows = 2 * seq
        s = jax.lax.broadcasted_iota(jnp.int32, (_TBL, crows), 1)
        k = jax.lax.broadcasted_iota(jnp.int32, (_TBL, crows), 0)
        idx = jnp.clip(seq + _MAX_LENGTH - 1 - s, 0, _TBL - 1)
        g = (k == idx).astype(jnp.float32)
        ctv[...] = jax.lax.dot_general(
            tbl_ref[...], g, (((0,), (0,)), ((), ())),
            precision=jax.lax.Precision.HIGHEST,
            preferred_element_type=jnp.float32)

    m = seq - 1 - pid
    a = m // 128
    p = m - a * 128
    hi_end = seq - _MAX_LENGTH
    lo_start = seq + _MAX_LENGTH - 1
    t_hi = ctv[:, 0:128].reshape(8, 8, 128)
    t_lo = ctv[:, 2 * seq - 128:2 * seq].reshape(8, 8, 128)

    for jt in range(seq // 128):
        lo0 = m + 128 * jt
        all_hi = lo0 + 127 <= hi_end
        all_lo = lo0 >= lo_start

        @pl.when(all_hi)
        def _hi():
            out_ref[0, :, jt] = t_hi

        @pl.when(all_lo)
        def _lo():
            out_ref[0, :, jt] = t_lo

        @pl.when(jnp.logical_not(jnp.logical_or(all_hi, all_lo)))
        def _band():
            w = ctv[:, pl.ds(pl.multiple_of((a + jt) * 128, 128), 256)]
            sl = pltpu.roll(w, (256 - p) % 256, 1)[:, 0:128]
            out_ref[0, :, jt] = sl.reshape(8, 8, 128)


def kernel(hidden_states, pe_k_weight):
    seq = hidden_states.shape[1]
    out = pl.pallas_call(
        functools.partial(_body, seq=seq),
        grid=(seq,),
        in_specs=[pl.BlockSpec((_TBL, _DIM), lambda i: (0, 0))],
        out_specs=pl.BlockSpec(
            (1, 8, seq // 128, 8, 128), lambda i: (i, 0, 0, 0, 0)),
        out_shape=jax.ShapeDtypeStruct(
            (seq, 8, seq // 128, 8, 128), jnp.float32),
        scratch_shapes=[
            pltpu.VMEM((_DIM, 2 * seq), jnp.float32),
        ],
        compiler_params=pltpu.CompilerParams(
            dimension_semantics=("arbitrary",)),
    )(pe_k_weight)
    return out.transpose(0, 2, 4, 1, 3).reshape(seq, seq, _DIM)
